# baseline (speedup 1.0000x reference)
import functools

import numpy as np

import jax
import jax.numpy as jnp
from jax import lax
from jax.experimental import pallas as pl
from jax.experimental.pallas import tpu as pltpu
from jax.experimental.pallas import tpu_sc as plsc

N = 10000
E = 320000
H = 128
NC = 2
NS = 16
C = 96
NCH = 105
PIECES = 5
PC = NCH // PIECES
CHUNKS_PAD = NCH * NC * NS
E_PAD = CHUNKS_PAD * C
NP = 10240
ROWS_PER_SUBCORE = NP // NS

_TC_R = 1000
_TC_GRID = N // _TC_R


def _sc_mesh():
    return plsc.VectorSubcoreMesh(core_axis_name="c", subcore_axis_name="s")


def _zero_spmem(z_hbm, table_sh, sid):
    base = sid * ROWS_PER_SUBCORE
    pltpu.sync_copy(z_hbm.at[pl.ds(base, ROWS_PER_SUBCORE)],
                    table_sh.at[pl.ds(base, ROWS_PER_SUBCORE)])


def _deg_phase(dst_hbm, ones_v, tab_sh, slabA, slabB, wid):
    pltpu.sync_copy(dst_hbm.at[wid, 0], slabA)
    for p in range(PIECES):
        cur, nxt = (slabA, slabB) if p % 2 == 0 else (slabB, slabA)
        if p + 1 < PIECES:
            pltpu.sync_copy(dst_hbm.at[wid, p + 1], nxt)

        @pl.loop(0, PC)
        def _(j):
            pltpu.sync_copy(ones_v, tab_sh.at[cur.at[j]], add=True)


def _agg_phase(x_hbm, src_hbm, dst_hbm, tab_sh, srcA, srcB, dstA, dstB,
               rows0, rows1, rows2, sem0, sem1, sem2, wid):
    bufs = ((rows0, sem0), (rows1, sem1), (rows2, sem2))

    def wait_gather(rows, sem):
        pltpu.make_async_copy(x_hbm.at[srcA.at[0]], rows, sem).wait()

    pltpu.sync_copy(src_hbm.at[wid, 0], srcA)
    pltpu.sync_copy(dst_hbm.at[wid, 0], dstA)
    for p in range(PIECES):
        src_c, dst_c = (srcA, dstA) if p % 2 == 0 else (srcB, dstB)
        src_n, dst_n = (srcB, dstB) if p % 2 == 0 else (srcA, dstA)
        for k, (rows, sem) in enumerate(bufs):
            pltpu.async_copy(x_hbm.at[src_c.at[k]], rows, sem)

        @pl.loop(0, PC - 3, step=3)
        def _(j):
            for k, (rows, sem) in enumerate(bufs):
                wait_gather(rows, sem)
                pltpu.sync_copy(rows, tab_sh.at[dst_c.at[j + k]], add=True)
                pltpu.async_copy(x_hbm.at[src_c.at[j + k + 3]], rows, sem)

        if p + 1 < PIECES:
            pltpu.sync_copy(src_hbm.at[wid, p + 1], src_n)
            pltpu.sync_copy(dst_hbm.at[wid, p + 1], dst_n)
        for k, (rows, sem) in enumerate(bufs):
            wait_gather(rows, sem)
            pltpu.sync_copy(rows, tab_sh.at[dst_c.at[PC - 3 + k]], add=True)


_SC_SCRATCH = [
    pltpu.VMEM((PC, C), jnp.int32),
    pltpu.VMEM((PC, C), jnp.int32),
    pltpu.VMEM((PC, C), jnp.int32),
    pltpu.VMEM((PC, C), jnp.int32),
    pltpu.VMEM((C, H), jnp.float32),
    pltpu.VMEM((C, H), jnp.float32),
    pltpu.VMEM((C, H), jnp.float32),
    pltpu.VMEM_SHARED((NP, H), jnp.float32),
    pltpu.SemaphoreType.DMA,
    pltpu.SemaphoreType.DMA,
    pltpu.SemaphoreType.DMA,
]


def _sc_deg_agg(x, src_slabs, dst_slabs, ones_src, zerosNP):
    @functools.partial(
        pl.kernel,
        mesh=_sc_mesh(),
        out_type=(jax.ShapeDtypeStruct((NC, NP, H), jnp.float32),
                  jax.ShapeDtypeStruct((NC, NP, H), jnp.float32)),
        scratch_types=_SC_SCRATCH,
    )
    def deg_agg_kernel(x_hbm, src_hbm, dst_hbm, ones_hbm, z_hbm, deg_hbm,
                       out_hbm, srcA, srcB, dstA, dstB, rows0, rows1, rows2,
                       tab_sh, sem0, sem1, sem2):
        cid = lax.axis_index("c")
        sid = lax.axis_index("s")
        wid = sid * NC + cid
        base = sid * ROWS_PER_SUBCORE
        pltpu.sync_copy(ones_hbm, rows0)
        _zero_spmem(z_hbm, tab_sh, sid)
        plsc.subcore_barrier()

        _deg_phase(dst_hbm, rows0, tab_sh, dstA, dstB, wid)

        plsc.subcore_barrier()
        pltpu.sync_copy(tab_sh.at[pl.ds(base, ROWS_PER_SUBCORE)],
                        deg_hbm.at[cid, pl.ds(base, ROWS_PER_SUBCORE)])
        _zero_spmem(z_hbm, tab_sh, sid)
        plsc.subcore_barrier()

        _agg_phase(x_hbm, src_hbm, dst_hbm, tab_sh, srcA, srcB, dstA, dstB,
                   rows0, rows1, rows2, sem0, sem1, sem2, wid)

        plsc.subcore_barrier()
        pltpu.sync_copy(tab_sh.at[pl.ds(base, ROWS_PER_SUBCORE)],
                        out_hbm.at[cid, pl.ds(base, ROWS_PER_SUBCORE)])

    return deg_agg_kernel(x, src_slabs, dst_slabs, ones_src, zerosNP)


def _sc_segment_sum(x, src_slabs, dst_slabs, zerosNP):
    @functools.partial(
        pl.kernel,
        mesh=_sc_mesh(),
        out_type=jax.ShapeDtypeStruct((NC, NP, H), jnp.float32),
        scratch_types=_SC_SCRATCH,
    )
    def agg_kernel(x_hbm, src_hbm, dst_hbm, z_hbm, out_hbm, srcA, srcB, dstA,
                   dstB, rows0, rows1, rows2, agg_sh, sem0, sem1, sem2):
        cid = lax.axis_index("c")
        sid = lax.axis_index("s")
        wid = sid * NC + cid
        _zero_spmem(z_hbm, agg_sh, sid)
        plsc.subcore_barrier()

        _agg_phase(x_hbm, src_hbm, dst_hbm, agg_sh, srcA, srcB, dstA, dstB,
                   rows0, rows1, rows2, sem0, sem1, sem2, wid)

        plsc.subcore_barrier()
        base = sid * ROWS_PER_SUBCORE
        pltpu.sync_copy(agg_sh.at[pl.ds(base, ROWS_PER_SUBCORE)],
                        out_hbm.at[cid, pl.ds(base, ROWS_PER_SUBCORE)])

    return agg_kernel(x, src_slabs, dst_slabs, zerosNP)


def _tc_ffn_ln(x, w1, b1, w2, b2, g, b):
    def body(x_ref, w1_ref, b1_ref, w2_ref, b2_ref, g_ref, b_ref, o_ref):
        xv = x_ref[...]
        h = jnp.maximum(_dot(xv, w1_ref) + b1_ref[...], 0.0)
        ff = _dot(h, w2_ref) + b2_ref[...]
        y = ff + xv
        mu = jnp.mean(y, axis=-1, keepdims=True)
        var = jnp.mean((y - mu) ** 2, axis=-1, keepdims=True)
        o_ref[...] = (y - mu) * lax.rsqrt(var + 1e-5) * g_ref[...] + b_ref[...]

    full = lambda i: (0, 0)
    return pl.pallas_call(
        body,
        grid=(_TC_GRID,),
        in_specs=[
            pl.BlockSpec((_TC_R, H), lambda i: (i, 0)),
            pl.BlockSpec((H, H), full),
            pl.BlockSpec((1, H), full),
            pl.BlockSpec((H, H), full),
            pl.BlockSpec((1, H), full),
            pl.BlockSpec((1, H), full),
            pl.BlockSpec((1, H), full),
        ],
        out_specs=pl.BlockSpec((_TC_R, H), lambda i: (i, 0)),
        out_shape=jax.ShapeDtypeStruct((N, H), jnp.float32),
    )(x, w1, b1, w2, b2, g, b)


def _dot(a, w_ref):
    return jnp.dot(a.astype(jnp.bfloat16), w_ref[...].astype(jnp.bfloat16),
                   preferred_element_type=jnp.float32)


def _ln(y, g, b):
    mu = jnp.mean(y, axis=-1, keepdims=True)
    var = jnp.mean((y - mu) ** 2, axis=-1, keepdims=True)
    return (y - mu) * lax.rsqrt(var + 1e-5) * g + b


def _gcn_ln_block(p_ref, d_ref, x_ref, w_ref, bias_ref, g_ref, b_ref):
    agg = p_ref[0] + p_ref[1]
    deg = d_ref[0, :, 0:1] + d_ref[1, :, 0:1]
    agg = agg / jnp.maximum(deg, 1.0)
    gcn = jnp.maximum(_dot(agg, w_ref) + bias_ref[...], 0.0)
    return _ln(gcn + x_ref[...], g_ref[...], b_ref[...])


def _tc_gcn_ln(partial, degp, x, w, bias, g, b):
    def body(p_ref, d_ref, x_ref, w_ref, bias_ref, g_ref, b_ref, o_ref):
        o_ref[...] = _gcn_ln_block(p_ref, d_ref, x_ref, w_ref, bias_ref,
                                   g_ref, b_ref)

    full = lambda i: (0, 0)
    return pl.pallas_call(
        body,
        grid=(_TC_GRID,),
        in_specs=[
            pl.BlockSpec((NC, _TC_R, H), lambda i: (0, i, 0)),
            pl.BlockSpec((NC, _TC_R, H), lambda i: (0, i, 0)),
            pl.BlockSpec((_TC_R, H), lambda i: (i, 0)),
            pl.BlockSpec((H, H), full),
            pl.BlockSpec((1, H), full),
            pl.BlockSpec((1, H), full),
            pl.BlockSpec((1, H), full),
        ],
        out_specs=pl.BlockSpec((_TC_R, H), lambda i: (i, 0)),
        out_shape=jax.ShapeDtypeStruct((N, H), jnp.float32),
    )(partial, degp, x, w, bias, g, b)


def _tc_gcn_ffn_ln(partial, degp, x, w, bias, g, b, w1, b1, w2, b2, g2, b2b):
    def body(p_ref, d_ref, x_ref, w_ref, bias_ref, g_ref, b_ref,
             w1_ref, b1_ref, w2_ref, b2_ref, g2_ref, b2b_ref, o_ref):
        y0 = _gcn_ln_block(p_ref, d_ref, x_ref, w_ref, bias_ref, g_ref, b_ref)
        hmid = jnp.maximum(_dot(y0, w1_ref) + b1_ref[...], 0.0)
        ff = _dot(hmid, w2_ref) + b2_ref[...]
        o_ref[...] = _ln(ff + y0, g2_ref[...], b2b_ref[...])

    full = lambda i: (0, 0)
    return pl.pallas_call(
        body,
        grid=(_TC_GRID,),
        in_specs=[
            pl.BlockSpec((NC, _TC_R, H), lambda i: (0, i, 0)),
            pl.BlockSpec((NC, _TC_R, H), lambda i: (0, i, 0)),
            pl.BlockSpec((_TC_R, H), lambda i: (i, 0)),
            pl.BlockSpec((H, H), full),
            pl.BlockSpec((1, H), full),
            pl.BlockSpec((1, H), full),
            pl.BlockSpec((1, H), full),
            pl.BlockSpec((H, H), full),
            pl.BlockSpec((1, H), full),
            pl.BlockSpec((H, H), full),
            pl.BlockSpec((1, H), full),
            pl.BlockSpec((1, H), full),
            pl.BlockSpec((1, H), full),
        ],
        out_specs=pl.BlockSpec((_TC_R, H), lambda i: (i, 0)),
        out_shape=jax.ShapeDtypeStruct((N, H), jnp.float32),
    )(partial, degp, x, w, bias, g, b, w1, b1, w2, b2, g2, b2b)


_PAD = E_PAD - E
_PAD_SRC = np.asarray((np.arange(_PAD) * 97) % N, np.int32)
_PAD_DST = np.asarray(N + (np.arange(_PAD) % (NP - N)), np.int32)


def kernel(features, edge_index, params):
    src = jnp.concatenate([edge_index[0].astype(jnp.int32), _PAD_SRC])
    dst = jnp.concatenate([edge_index[1].astype(jnp.int32), _PAD_DST])
    src3 = src.reshape(NC * NS, PIECES, PC, C)
    dst3 = dst.reshape(NC * NS, PIECES, PC, C)
    onesH = jnp.ones((C, H), jnp.float32)
    zerosNP = jnp.zeros((NP, H), jnp.float32)

    p0 = params["l0"]
    p1 = params["l1"]
    r = lambda v: v.reshape(1, H)

    out0 = _tc_ffn_ln(features, p0["w1"], r(p0["b1"]), p0["w2"], r(p0["b2"]),
                      r(p0["ln1_g"]), r(p0["ln1_b"]))
    degp, part0 = _sc_deg_agg(out0, src3, dst3, onesH, zerosNP)
    out1 = _tc_gcn_ffn_ln(part0, degp, out0, p0["gcn_w"], r(p0["gcn_b"]),
                          r(p0["ln2_g"]), r(p0["ln2_b"]),
                          p1["w1"], r(p1["b1"]), p1["w2"], r(p1["b2"]),
                          r(p1["ln1_g"]), r(p1["ln1_b"]))
    part1 = _sc_segment_sum(out1, src3, dst3, zerosNP)
    return _tc_gcn_ln(part1, degp, out1, p1["gcn_w"], r(p1["gcn_b"]),
                      r(p1["ln2_g"]), r(p1["ln2_b"]))


# speedup vs baseline: 1.0034x; 1.0034x over previous
import functools

import numpy as np

import jax
import jax.numpy as jnp
from jax import lax
from jax.experimental import pallas as pl
from jax.experimental.pallas import tpu as pltpu
from jax.experimental.pallas import tpu_sc as plsc

N = 10000
E = 320000
H = 128
NC = 2
NS = 16
C = 96
NCH = 105
PIECES = 5
PC = NCH // PIECES
CHUNKS_PAD = NCH * NC * NS
E_PAD = CHUNKS_PAD * C
NP = 10240
ROWS_PER_SUBCORE = NP // NS

_TC_R = 1000
_TC_GRID = N // _TC_R


def _sc_mesh():
    return plsc.VectorSubcoreMesh(core_axis_name="c", subcore_axis_name="s")


def _zero_spmem(z_hbm, table_sh, sid):
    base = sid * ROWS_PER_SUBCORE
    pltpu.sync_copy(z_hbm.at[pl.ds(base, ROWS_PER_SUBCORE)],
                    table_sh.at[pl.ds(base, ROWS_PER_SUBCORE)])


def _deg_phase(dst_hbm, ones_v, tab_sh, slabA, slabB, wid):
    pltpu.sync_copy(dst_hbm.at[wid, 0], slabA)
    for p in range(PIECES):
        cur, nxt = (slabA, slabB) if p % 2 == 0 else (slabB, slabA)
        if p + 1 < PIECES:
            pltpu.sync_copy(dst_hbm.at[wid, p + 1], nxt)

        @pl.loop(0, PC)
        def _(j):
            pltpu.sync_copy(ones_v, tab_sh.at[cur.at[j]], add=True)


def _agg_phase(x_hbm, src_hbm, dst_hbm, tab_sh, srcA, srcB, dstA, dstB,
               rows0, rows1, rows2, sem0, sem1, sem2, wid):
    bufs = ((rows0, sem0), (rows1, sem1), (rows2, sem2))

    def wait_gather(rows, sem):
        pltpu.make_async_copy(x_hbm.at[srcA.at[0]], rows, sem).wait()

    pltpu.sync_copy(src_hbm.at[wid, 0], srcA)
    pltpu.sync_copy(dst_hbm.at[wid, 0], dstA)
    for p in range(PIECES):
        src_c, dst_c = (srcA, dstA) if p % 2 == 0 else (srcB, dstB)
        src_n, dst_n = (srcB, dstB) if p % 2 == 0 else (srcA, dstA)
        for k, (rows, sem) in enumerate(bufs):
            pltpu.async_copy(x_hbm.at[src_c.at[k]], rows, sem)

        @pl.loop(0, PC - 3, step=3)
        def _(j):
            for k, (rows, sem) in enumerate(bufs):
                wait_gather(rows, sem)
                pltpu.sync_copy(rows, tab_sh.at[dst_c.at[j + k]], add=True)
                pltpu.async_copy(x_hbm.at[src_c.at[j + k + 3]], rows, sem)

        if p + 1 < PIECES:
            pltpu.sync_copy(src_hbm.at[wid, p + 1], src_n)
            pltpu.sync_copy(dst_hbm.at[wid, p + 1], dst_n)
        for k, (rows, sem) in enumerate(bufs):
            wait_gather(rows, sem)
            pltpu.sync_copy(rows, tab_sh.at[dst_c.at[PC - 3 + k]], add=True)


_SC_SCRATCH = [
    pltpu.VMEM((PC, C), jnp.int32),
    pltpu.VMEM((PC, C), jnp.int32),
    pltpu.VMEM((PC, C), jnp.int32),
    pltpu.VMEM((PC, C), jnp.int32),
    pltpu.VMEM((C, H), jnp.float32),
    pltpu.VMEM((C, H), jnp.float32),
    pltpu.VMEM((C, H), jnp.float32),
    pltpu.VMEM_SHARED((NP, H), jnp.float32),
    pltpu.SemaphoreType.DMA,
    pltpu.SemaphoreType.DMA,
    pltpu.SemaphoreType.DMA,
]


def _sc_deg_agg(x, src_slabs, dst_slabs, ones_src, zerosNP):
    @functools.partial(
        pl.kernel,
        mesh=_sc_mesh(),
        out_type=(jax.ShapeDtypeStruct((NC, NP, H), jnp.float32),
                  jax.ShapeDtypeStruct((NC, NP, H), jnp.float32)),
        scratch_types=_SC_SCRATCH,
    )
    def deg_agg_kernel(x_hbm, src_hbm, dst_hbm, ones_hbm, z_hbm, deg_hbm,
                       out_hbm, srcA, srcB, dstA, dstB, rows0, rows1, rows2,
                       tab_sh, sem0, sem1, sem2):
        cid = lax.axis_index("c")
        sid = lax.axis_index("s")
        wid = sid * NC + cid
        base = sid * ROWS_PER_SUBCORE
        pltpu.sync_copy(ones_hbm, rows0)
        _zero_spmem(z_hbm, tab_sh, sid)
        plsc.subcore_barrier()

        _deg_phase(dst_hbm, rows0, tab_sh, dstA, dstB, wid)

        plsc.subcore_barrier()
        pltpu.sync_copy(tab_sh.at[pl.ds(base, ROWS_PER_SUBCORE)],
                        deg_hbm.at[cid, pl.ds(base, ROWS_PER_SUBCORE)])
        _zero_spmem(z_hbm, tab_sh, sid)
        plsc.subcore_barrier()

        _agg_phase(x_hbm, src_hbm, dst_hbm, tab_sh, srcA, srcB, dstA, dstB,
                   rows0, rows1, rows2, sem0, sem1, sem2, wid)

        plsc.subcore_barrier()
        pltpu.sync_copy(tab_sh.at[pl.ds(base, ROWS_PER_SUBCORE)],
                        out_hbm.at[cid, pl.ds(base, ROWS_PER_SUBCORE)])

    return deg_agg_kernel(x, src_slabs, dst_slabs, ones_src, zerosNP)


def _sc_segment_sum(x, src_slabs, dst_slabs, zerosNP):
    @functools.partial(
        pl.kernel,
        mesh=_sc_mesh(),
        out_type=jax.ShapeDtypeStruct((NC, NP, H), jnp.float32),
        scratch_types=_SC_SCRATCH,
    )
    def agg_kernel(x_hbm, src_hbm, dst_hbm, z_hbm, out_hbm, srcA, srcB, dstA,
                   dstB, rows0, rows1, rows2, agg_sh, sem0, sem1, sem2):
        cid = lax.axis_index("c")
        sid = lax.axis_index("s")
        wid = sid * NC + cid
        _zero_spmem(z_hbm, agg_sh, sid)
        plsc.subcore_barrier()

        _agg_phase(x_hbm, src_hbm, dst_hbm, agg_sh, srcA, srcB, dstA, dstB,
                   rows0, rows1, rows2, sem0, sem1, sem2, wid)

        plsc.subcore_barrier()
        base = sid * ROWS_PER_SUBCORE
        pltpu.sync_copy(agg_sh.at[pl.ds(base, ROWS_PER_SUBCORE)],
                        out_hbm.at[cid, pl.ds(base, ROWS_PER_SUBCORE)])

    return agg_kernel(x, src_slabs, dst_slabs, zerosNP)


def _tc_ffn_ln(x, w1, b1, w2, b2, g, b):
    def body(x_ref, w1_ref, b1_ref, w2_ref, b2_ref, g_ref, b_ref, o_ref):
        xv = x_ref[...]
        h = jnp.maximum(_dot(xv, w1_ref) + b1_ref[...], 0.0)
        ff = _dot(h, w2_ref) + b2_ref[...]
        y = ff + xv
        mu = jnp.mean(y, axis=-1, keepdims=True)
        var = jnp.mean((y - mu) ** 2, axis=-1, keepdims=True)
        o_ref[...] = (y - mu) * lax.rsqrt(var + 1e-5) * g_ref[...] + b_ref[...]

    full = lambda i: (0, 0)
    return pl.pallas_call(
        body,
        grid=(_TC_GRID,),
        in_specs=[
            pl.BlockSpec((_TC_R, H), lambda i: (i, 0)),
            pl.BlockSpec((H, H), full),
            pl.BlockSpec((1, H), full),
            pl.BlockSpec((H, H), full),
            pl.BlockSpec((1, H), full),
            pl.BlockSpec((1, H), full),
            pl.BlockSpec((1, H), full),
        ],
        out_specs=pl.BlockSpec((_TC_R, H), lambda i: (i, 0)),
        out_shape=jax.ShapeDtypeStruct((N, H), jnp.float32),
    )(x, w1, b1, w2, b2, g, b)


def _dot(a, w_ref):
    return jnp.dot(a, w_ref[...], preferred_element_type=jnp.float32)


def _ln(y, g, b):
    mu = jnp.mean(y, axis=-1, keepdims=True)
    var = jnp.mean((y - mu) ** 2, axis=-1, keepdims=True)
    return (y - mu) * lax.rsqrt(var + 1e-5) * g + b


def _gcn_ln_block(p_ref, d_ref, x_ref, w_ref, bias_ref, g_ref, b_ref):
    agg = p_ref[0] + p_ref[1]
    deg = d_ref[0, :, 0:1] + d_ref[1, :, 0:1]
    agg = agg / jnp.maximum(deg, 1.0)
    gcn = jnp.maximum(_dot(agg, w_ref) + bias_ref[...], 0.0)
    return _ln(gcn + x_ref[...], g_ref[...], b_ref[...])


def _tc_gcn_ln(partial, degp, x, w, bias, g, b):
    def body(p_ref, d_ref, x_ref, w_ref, bias_ref, g_ref, b_ref, o_ref):
        o_ref[...] = _gcn_ln_block(p_ref, d_ref, x_ref, w_ref, bias_ref,
                                   g_ref, b_ref)

    full = lambda i: (0, 0)
    return pl.pallas_call(
        body,
        grid=(_TC_GRID,),
        in_specs=[
            pl.BlockSpec((NC, _TC_R, H), lambda i: (0, i, 0)),
            pl.BlockSpec((NC, _TC_R, H), lambda i: (0, i, 0)),
            pl.BlockSpec((_TC_R, H), lambda i: (i, 0)),
            pl.BlockSpec((H, H), full),
            pl.BlockSpec((1, H), full),
            pl.BlockSpec((1, H), full),
            pl.BlockSpec((1, H), full),
        ],
        out_specs=pl.BlockSpec((_TC_R, H), lambda i: (i, 0)),
        out_shape=jax.ShapeDtypeStruct((N, H), jnp.float32),
    )(partial, degp, x, w, bias, g, b)


def _tc_gcn_ffn_ln(partial, degp, x, w, bias, g, b, w1, b1, w2, b2, g2, b2b):
    def body(p_ref, d_ref, x_ref, w_ref, bias_ref, g_ref, b_ref,
             w1_ref, b1_ref, w2_ref, b2_ref, g2_ref, b2b_ref, o_ref):
        y0 = _gcn_ln_block(p_ref, d_ref, x_ref, w_ref, bias_ref, g_ref, b_ref)
        hmid = jnp.maximum(_dot(y0, w1_ref) + b1_ref[...], 0.0)
        ff = _dot(hmid, w2_ref) + b2_ref[...]
        o_ref[...] = _ln(ff + y0, g2_ref[...], b2b_ref[...])

    full = lambda i: (0, 0)
    return pl.pallas_call(
        body,
        grid=(_TC_GRID,),
        in_specs=[
            pl.BlockSpec((NC, _TC_R, H), lambda i: (0, i, 0)),
            pl.BlockSpec((NC, _TC_R, H), lambda i: (0, i, 0)),
            pl.BlockSpec((_TC_R, H), lambda i: (i, 0)),
            pl.BlockSpec((H, H), full),
            pl.BlockSpec((1, H), full),
            pl.BlockSpec((1, H), full),
            pl.BlockSpec((1, H), full),
            pl.BlockSpec((H, H), full),
            pl.BlockSpec((1, H), full),
            pl.BlockSpec((H, H), full),
            pl.BlockSpec((1, H), full),
            pl.BlockSpec((1, H), full),
            pl.BlockSpec((1, H), full),
        ],
        out_specs=pl.BlockSpec((_TC_R, H), lambda i: (i, 0)),
        out_shape=jax.ShapeDtypeStruct((N, H), jnp.float32),
    )(partial, degp, x, w, bias, g, b, w1, b1, w2, b2, g2, b2b)


_PAD = E_PAD - E
_PAD_SRC = np.asarray((np.arange(_PAD) * 97) % N, np.int32)
_PAD_DST = np.asarray(N + (np.arange(_PAD) % (NP - N)), np.int32)


def kernel(features, edge_index, params):
    src = jnp.concatenate([edge_index[0].astype(jnp.int32), _PAD_SRC])
    dst = jnp.concatenate([edge_index[1].astype(jnp.int32), _PAD_DST])
    src3 = src.reshape(NC * NS, PIECES, PC, C)
    dst3 = dst.reshape(NC * NS, PIECES, PC, C)
    onesH = jnp.ones((C, H), jnp.float32)
    zerosNP = jnp.zeros((NP, H), jnp.float32)

    p0 = params["l0"]
    p1 = params["l1"]
    r = lambda v: v.reshape(1, H)

    out0 = _tc_ffn_ln(features, p0["w1"], r(p0["b1"]), p0["w2"], r(p0["b2"]),
                      r(p0["ln1_g"]), r(p0["ln1_b"]))
    degp, part0 = _sc_deg_agg(out0, src3, dst3, onesH, zerosNP)
    out1 = _tc_gcn_ffn_ln(part0, degp, out0, p0["gcn_w"], r(p0["gcn_b"]),
                          r(p0["ln2_g"]), r(p0["ln2_b"]),
                          p1["w1"], r(p1["b1"]), p1["w2"], r(p1["b2"]),
                          r(p1["ln1_g"]), r(p1["ln1_b"]))
    part1 = _sc_segment_sum(out1, src3, dst3, zerosNP)
    return _tc_gcn_ln(part1, degp, out1, p1["gcn_w"], r(p1["gcn_b"]),
                      r(p1["ln2_g"]), r(p1["ln2_b"]))
